# baseline (speedup 1.0000x reference)
import jax
import jax.numpy as jnp
from jax import lax
from jax.experimental import pallas as pl
from jax.experimental.pallas import tpu as pltpu
from jax.experimental.pallas import tpu_sc as plsc

_D = 32
_B = 16384
_PACK = 4
_DW = _D * _PACK
_CHUNK = 128

_info = plsc.get_sparse_core_info()
_NC, _NS = _info.num_cores, _info.num_subcores
_NW = _NC * _NS
_BPW = _B // _NW
_NCH = _BPW // _CHUNK


def _gather_body(table_hbm, idx_hbm, out_hbm, idx_v, line_v, lines_g,
                 rows_o, sem):
    wid = lax.axis_index("s") * _NC + lax.axis_index("c")
    base = wid * _BPW
    pltpu.sync_copy(idx_hbm.at[pl.ds(pl.multiple_of(base, 8), _BPW)], idx_v)

    def to_lines(i, carry):
        v = idx_v[pl.ds(i * 16, 16)]
        line_v[pl.ds(i * 16, 16)] = lax.shift_right_logical(v, 2)
        return carry

    lax.fori_loop(0, _BPW // 16, to_lines, 0)

    for ch in range(_NCH):
        pltpu.make_async_copy(
            table_hbm.at[line_v.at[pl.ds(ch * _CHUNK, _CHUNK)]],
            lines_g.at[pl.ds(ch * _CHUNK, _CHUNK), :],
            sem,
        ).start()
    pltpu.make_async_copy(
        table_hbm.at[pl.ds(0, _BPW)], lines_g, sem
    ).wait()

    def extract(i, carry):
        v = idx_v[pl.ds(i * 16, 16)]
        cv = lax.shift_left(lax.bitwise_and(v, _PACK - 1), 5)
        for j in range(16):
            k = i * 16 + j
            c = pl.multiple_of(cv[j], 32)
            rows_o[pl.ds(k * _D, 16)] = lines_g[k, pl.ds(c, 16)]
            rows_o[pl.ds(k * _D + 16, 16)] = lines_g[k, pl.ds(c + 16, 16)]
        return carry

    lax.fori_loop(0, _BPW // 16, extract, 0)
    pltpu.sync_copy(
        rows_o,
        out_hbm.at[pl.ds(pl.multiple_of(base * _D, 8), _BPW * _D)],
    )


@jax.jit
def kernel(node_vecs, country_idx):
    table = node_vecs.reshape(-1, _DW)
    idx = country_idx.astype(jnp.int32).reshape(_B)
    mesh = plsc.VectorSubcoreMesh(core_axis_name="c", subcore_axis_name="s")
    f = pl.kernel(
        _gather_body,
        mesh=mesh,
        out_type=jax.ShapeDtypeStruct((_B * _D,), jnp.float32),
        scratch_types=[
            pltpu.VMEM((_BPW,), jnp.int32),
            pltpu.VMEM((_BPW,), jnp.int32),
            pltpu.VMEM((_BPW, _DW), jnp.float32),
            pltpu.VMEM((_BPW * _D,), jnp.float32),
            pltpu.SemaphoreType.DMA,
        ],
        compiler_params=pltpu.CompilerParams(
            skip_device_barrier=True,
            disable_semaphore_checks=True,
            disable_bounds_checks=True,
        ),
    )
    return f(table, idx).reshape(_B, _D)


# speedup vs baseline: 1.6732x; 1.6732x over previous
import jax
import jax.numpy as jnp
from jax import lax
from jax.experimental import pallas as pl
from jax.experimental.pallas import tpu as pltpu
from jax.experimental.pallas import tpu_sc as plsc

_D = 32
_B = 16384

_info = plsc.get_sparse_core_info()
_NC, _NS = _info.num_cores, _info.num_subcores
_NW = _NC * _NS
_BPW = _B // _NW


def _gather_body(table_hbm, idx_hbm, out_hbm, idx_v, rows_v, sem):
    wid = lax.axis_index("s") * _NC + lax.axis_index("c")
    base = wid * _BPW
    pltpu.sync_copy(idx_hbm.at[wid], idx_v)

    def step(i, carry):
        v = idx_v[pl.ds(i * 16, 16)]
        for j in range(16):
            pltpu.make_async_copy(
                table_hbm.at[pl.ds(v[j], 1)],
                rows_v.at[pl.ds(i * 16 + j, 1)],
                sem,
            ).start()
        return carry

    lax.fori_loop(0, _BPW // 16, step, 0)
    pltpu.make_async_copy(table_hbm.at[pl.ds(0, _BPW)], rows_v, sem).wait()
    pltpu.sync_copy(rows_v, out_hbm.at[pl.ds(base, _BPW)])


@jax.jit
def kernel(node_vecs, country_idx):
    idx = country_idx.astype(jnp.int32).reshape(_NW, _BPW)
    mesh = plsc.VectorSubcoreMesh(core_axis_name="c", subcore_axis_name="s")
    f = pl.kernel(
        _gather_body,
        mesh=mesh,
        out_type=jax.ShapeDtypeStruct((_B, _D), jnp.float32),
        scratch_types=[
            pltpu.VMEM((_BPW,), jnp.int32),
            pltpu.VMEM((_BPW, _D), jnp.float32),
            pltpu.SemaphoreType.DMA,
        ],
        compiler_params=pltpu.CompilerParams(
            skip_device_barrier=True,
            disable_semaphore_checks=True,
            disable_bounds_checks=True,
        ),
    )
    return f(node_vecs, idx)


# speedup vs baseline: 3.4523x; 2.0634x over previous
import jax
import jax.numpy as jnp
from jax import lax
from jax.experimental import pallas as pl
from jax.experimental.pallas import tpu as pltpu
from jax.experimental.pallas import tpu_sc as plsc

_D = 32
_B = 16384
_TW = 128

_info = plsc.get_sparse_core_info()
_NC, _NS = _info.num_cores, _info.num_subcores
_NW = _NC * _NS
_BPW = _B // _NW
_NT = 16
_NR = _BPW // _NT


def _gather_body(table_hbm, idx_hbm, out_hbm, idx_v, tb, slab, sem):
    wid = lax.axis_index("s") * _NC + lax.axis_index("c")
    base = wid * _BPW
    pltpu.sync_copy(idx_hbm.at[pl.ds(pl.multiple_of(base, 8), _BPW)], idx_v)
    f_lo = lax.iota(jnp.int32, 16)
    f_hi = f_lo + 16

    def round_fn(g, carry):
        v = idx_v[pl.ds(g * _NT, _NT)]
        tv = lax.shift_left(lax.shift_right_logical(v, 7), 7)
        cv = lax.bitwise_and(v, _TW - 1)
        for s in range(_NT):
            pltpu.make_async_copy(
                table_hbm.at[:, pl.ds(pl.multiple_of(tv[s], _TW), _TW)],
                tb.at[s],
                sem,
            ).start()
        for s in range(_NT):
            pltpu.make_async_copy(
                table_hbm.at[:, pl.ds(0, _TW)], tb.at[s], sem
            ).wait()
        for s in range(_NT):
            c16 = jnp.full((16,), cv[s], jnp.int32)
            a = plsc.load_gather(tb.at[s], [f_lo, c16])
            b = plsc.load_gather(tb.at[s], [f_hi, c16])
            k = g * _NT + s
            slab[pl.ds(k * _D, 16)] = a
            slab[pl.ds(k * _D + 16, 16)] = b
        return carry

    lax.fori_loop(0, _NR, round_fn, 0)
    pltpu.sync_copy(
        slab, out_hbm.at[pl.ds(pl.multiple_of(base * _D, 8), _BPW * _D)]
    )


@jax.jit
def kernel(node_vecs, country_idx):
    table_t = node_vecs.T
    idx = country_idx.astype(jnp.int32).reshape(_B)
    mesh = plsc.VectorSubcoreMesh(core_axis_name="c", subcore_axis_name="s")
    f = pl.kernel(
        _gather_body,
        mesh=mesh,
        out_type=jax.ShapeDtypeStruct((_B * _D,), jnp.float32),
        scratch_types=[
            pltpu.VMEM((_BPW,), jnp.int32),
            pltpu.VMEM((_NT, _D, _TW), jnp.float32),
            pltpu.VMEM((_BPW * _D,), jnp.float32),
            pltpu.SemaphoreType.DMA,
        ],
        compiler_params=pltpu.CompilerParams(
            skip_device_barrier=True,
            disable_semaphore_checks=True,
            disable_bounds_checks=True,
            needs_layout_passes=False,
        ),
    )
    return f(table_t, idx).reshape(_B, _D)


# speedup vs baseline: 3.5842x; 1.0382x over previous
import jax
import jax.numpy as jnp
from jax import lax
from jax.experimental import pallas as pl
from jax.experimental.pallas import tpu as pltpu
from jax.experimental.pallas import tpu_sc as plsc

_D = 32
_B = 16384
_TW = 128

_info = plsc.get_sparse_core_info()
_NC, _NS = _info.num_cores, _info.num_subcores
_NW = _NC * _NS
_BPW = _B // _NW
_NT = 8
_NJ = _BPW // (2 * _NT)


def _gather_body(table_hbm, idx_hbm, out_hbm, idx_v, tba, tbb, slab,
                 sema, semb):
    wid = lax.axis_index("s") * _NC + lax.axis_index("c")
    base = wid * _BPW
    pltpu.sync_copy(idx_hbm.at[pl.ds(pl.multiple_of(base, 8), _BPW)], idx_v)
    f_lo = lax.iota(jnp.int32, 16)
    f_hi = f_lo + 16

    def enqueue(tv, lo, tb, sem):
        for s in range(_NT):
            pltpu.make_async_copy(
                table_hbm.at[:, pl.ds(pl.multiple_of(tv[lo + s], _TW), _TW)],
                tb.at[s],
                sem,
            ).start()

    def drain_extract(cv, lo, k0, tb, sem):
        for s in range(_NT):
            pltpu.make_async_copy(
                table_hbm.at[:, pl.ds(0, _TW)], tb.at[s], sem
            ).wait()
        for s in range(_NT):
            c16 = jnp.full((16,), cv[lo + s], jnp.int32)
            a = plsc.load_gather(tb.at[s], [f_lo, c16])
            b = plsc.load_gather(tb.at[s], [f_hi, c16])
            k = k0 + s
            slab[pl.ds(k * _D, 16)] = a
            slab[pl.ds(k * _D + 16, 16)] = b

    v0 = idx_v[pl.ds(0, 16)]
    tv0 = lax.shift_left(lax.shift_right_logical(v0, 7), 7)
    enqueue(tv0, 0, tba, sema)
    enqueue(tv0, _NT, tbb, semb)

    def body(j, carry):
        v = idx_v[pl.ds(j * 16, 16)]
        cv = lax.bitwise_and(v, _TW - 1)
        drain_extract(cv, 0, j * 16, tba, sema)

        @pl.when(j < _NJ - 1)
        def _():
            vn = idx_v[pl.ds((j + 1) * 16, 16)]
            tvn = lax.shift_left(lax.shift_right_logical(vn, 7), 7)
            enqueue(tvn, 0, tba, sema)

        drain_extract(cv, _NT, j * 16 + _NT, tbb, semb)

        @pl.when(j < _NJ - 1)
        def _():
            vn = idx_v[pl.ds((j + 1) * 16, 16)]
            tvn = lax.shift_left(lax.shift_right_logical(vn, 7), 7)
            enqueue(tvn, _NT, tbb, semb)

        return carry

    lax.fori_loop(0, _NJ, body, 0)
    pltpu.sync_copy(
        slab, out_hbm.at[pl.ds(pl.multiple_of(base * _D, 8), _BPW * _D)]
    )


@jax.jit
def kernel(node_vecs, country_idx):
    table_t = node_vecs.T
    idx = country_idx.astype(jnp.int32).reshape(_B)
    mesh = plsc.VectorSubcoreMesh(core_axis_name="c", subcore_axis_name="s")
    f = pl.kernel(
        _gather_body,
        mesh=mesh,
        out_type=jax.ShapeDtypeStruct((_B * _D,), jnp.float32),
        scratch_types=[
            pltpu.VMEM((_BPW,), jnp.int32),
            pltpu.VMEM((_NT, _D, _TW), jnp.float32),
            pltpu.VMEM((_NT, _D, _TW), jnp.float32),
            pltpu.VMEM((_BPW * _D,), jnp.float32),
            pltpu.SemaphoreType.DMA,
            pltpu.SemaphoreType.DMA,
        ],
        compiler_params=pltpu.CompilerParams(
            skip_device_barrier=True,
            disable_semaphore_checks=True,
            disable_bounds_checks=True,
            needs_layout_passes=False,
        ),
    )
    return f(table_t, idx).reshape(_B, _D)
